# baseline (speedup 1.0000x reference)
import functools

import jax
import jax.numpy as jnp
from jax import lax
from jax.experimental import pallas as pl
from jax.experimental.pallas import tpu as pltpu
from jax.experimental.pallas import tpu_sc as plsc

VOCAB = 1000000
DIM = 64
B = 4096
L = 200
EPS = 1e-12

NC = 2
NS = 16
LANES = 16
NW = NC * NS
BTILE = B // NW
PADW = 128
KV = DIM // LANES
CT = DIM // 8
UNROLL = 4
LPU = 4
TPU_ = LPU * BTILE
NU = L // LPU
OSTR = BTILE + 1


def _i16(v):
    return jnp.full((LANES,), v, jnp.int32)


def _rsqrt(x):
    i = lax.bitcast_convert_type(x, jnp.int32)
    y = lax.bitcast_convert_type(
        jnp.int32(0x5F3759DF) - lax.shift_right_logical(i, 1), jnp.float32)
    for _ in range(3):
        y = y * (1.5 - 0.5 * x * y * y)
    return y


def _sc_body(idsT_hbm, table_hbm, w_hbm, b_hbm, out_hbm,
             rows0, rows1, outt, icol0, icol1,
             w_v, b_v, sem_g0, sem_g1, sem_o):
    wkr = lax.axis_index("s") * NC + lax.axis_index("c")

    pltpu.sync_copy(w_hbm, w_v)
    pltpu.sync_copy(b_hbm, b_v)

    iota = lax.iota(jnp.int32, LANES)
    inv_dim = jnp.float32(1.0 / DIM)
    ct_half = lax.shift_right_logical(iota, 3)
    cc_lane = lax.bitwise_and(iota, _i16(7))

    def stage_idx(u, icol):
        pltpu.sync_copy(
            idsT_hbm.at[pl.ds(LPU * u, LPU), pl.ds(wkr * BTILE, BTILE)],
            icol)
        for j in range(LPU * BTILE // LANES):
            half = j // (BTILE // LANES)
            off = (j % (BTILE // LANES)) * LANES
            v = icol[half, pl.ds(off, LANES)]
            icol[half, pl.ds(off, LANES)] = v + v

    def fire_gather(icol, rows, sem):
        for half in range(LPU):
            pltpu.async_copy(table_hbm.at[icol.at[half]],
                             rows.at[pl.ds(half * BTILE, BTILE)], sem)

    def wait_gather(rows, sem):
        pltpu.make_async_copy(table_hbm.at[pl.ds(0, TPU_)], rows, sem).wait()

    def compute(rows):
        wb = ([w_v[pl.ds(k * LANES, LANES)] for k in range(KV)]
              + [b_v[pl.ds(k * LANES, LANES)] for k in range(KV)])

        def norm_body(un, wb):
            for tt in range(UNROLL):
                t = un * UNROLL + tt
                li = lax.shift_right_logical(t, 7)
                tb = lax.bitwise_and(t, BTILE - 1)
                vs = [rows[t, pl.ds(k * LANES, LANES)] for k in range(KV)]
                s = (vs[0] + vs[1]) + (vs[2] + vs[3])
                sq = (vs[0] * vs[0] + vs[1] * vs[1]) + (vs[2] * vs[2]
                                                        + vs[3] * vs[3])
                mean = jnp.sum(s) * inv_dim
                msq = jnp.sum(sq) * inv_dim
                var = msq - mean * mean
                rstd = _rsqrt(jnp.maximum(var, 0.0) + jnp.float32(EPS))
                c = -(mean * rstd)
                lv = jnp.zeros((LANES,), jnp.int32) + li
                tv = jnp.zeros((LANES,), jnp.int32) + tb
                for k in range(KV):
                    o = (vs[k] * rstd + c) * wb[k] + wb[KV + k]
                    plsc.store_scatter(
                        outt, [lv, 2 * k + ct_half, cc_lane, tv], o)
            return wb

        lax.fori_loop(0, TPU_ // UNROLL, norm_body, tuple(wb))

    def fire_out(u, sem):
        pltpu.async_copy(outt.at[:, :, :, pl.ds(0, BTILE)],
                         out_hbm.at[pl.ds(LPU * u, LPU), :, wkr], sem)

    def wait_out(sem):
        pltpu.make_async_copy(outt.at[:, :, :, pl.ds(0, BTILE)],
                              out_hbm.at[pl.ds(0, LPU), :, wkr], sem).wait()

    stage_idx(0, icol0)
    fire_gather(icol0, rows0, sem_g0)
    stage_idx(1, icol1)
    fire_gather(icol1, rows1, sem_g1)

    def body(h, _):
        u0 = 2 * h
        u1 = 2 * h + 1

        wait_gather(rows0, sem_g0)

        @pl.when(h > 0)
        def _():
            wait_out(sem_o)
        compute(rows0)
        fire_out(u0, sem_o)

        @pl.when(h < NU // 2 - 1)
        def _():
            stage_idx(u0 + 2, icol0)
            fire_gather(icol0, rows0, sem_g0)

        wait_gather(rows1, sem_g1)
        wait_out(sem_o)
        compute(rows1)
        fire_out(u1, sem_o)

        @pl.when(h < NU // 2 - 1)
        def _():
            stage_idx(u1 + 2, icol1)
            fire_gather(icol1, rows1, sem_g1)
        return 0

    lax.fori_loop(0, NU // 2, body, 0)
    wait_out(sem_o)


@jax.jit
def _sc_embed_ln(idsT, table_view, ln_weight, ln_bias):
    mesh = plsc.VectorSubcoreMesh(
        core_axis_name="c", subcore_axis_name="s",
        num_cores=NC, num_subcores=NS)
    return pl.kernel(
        _sc_body,
        out_type=jax.ShapeDtypeStruct((L, CT, NW, 8, 128), jnp.float32),
        mesh=mesh,
        compiler_params=pltpu.CompilerParams(
            needs_layout_passes=False, use_tc_tiling_on_sc=False),
        scratch_types=[
            pltpu.VMEM((TPU_, DIM), jnp.float32),
            pltpu.VMEM((TPU_, DIM), jnp.float32),
            pltpu.VMEM((LPU, CT, 8, OSTR), jnp.float32),
            pltpu.VMEM((LPU, BTILE), jnp.int32),
            pltpu.VMEM((LPU, BTILE), jnp.int32),
            pltpu.VMEM((DIM,), jnp.float32),
            pltpu.VMEM((DIM,), jnp.float32),
            pltpu.SemaphoreType.DMA,
            pltpu.SemaphoreType.DMA,
            pltpu.SemaphoreType.DMA,
        ],
    )(idsT, table_view, ln_weight, ln_bias)


def kernel(input_ids, table, ln_weight, ln_bias):
    table_pad = jnp.pad(table, ((0, 0), (0, PADW - DIM)))
    table_view = table_pad.reshape(2 * VOCAB, DIM)
    idsT = input_ids.astype(jnp.int32).T
    out5 = _sc_embed_ln(idsT, table_view, ln_weight, ln_bias)
    return out5.transpose(2, 4, 0, 1, 3).reshape(B, L, DIM)


# speedup vs baseline: 1.3712x; 1.3712x over previous
import functools

import jax
import jax.numpy as jnp
from jax import lax
from jax.experimental import pallas as pl
from jax.experimental.pallas import tpu as pltpu
from jax.experimental.pallas import tpu_sc as plsc

VOCAB = 1000000
DIM = 64
B = 4096
L = 200
EPS = 1e-12

NC = 2
NS = 16
LANES = 16
NW = NC * NS
TOK = B * L
TPW = TOK // NW
CHUNK = 512
GATHER = 128
KG = CHUNK // GATHER
NCHUNK = TPW // CHUNK
IDS_MINOR = 128
PADW = 128
KV = DIM // LANES


def _rsqrt(x):
    i = lax.bitcast_convert_type(x, jnp.int32)
    y = lax.bitcast_convert_type(
        jnp.int32(0x5F3759DF) - lax.shift_right_logical(i, 1), jnp.float32)
    for _ in range(3):
        y = y * (1.5 - 0.5 * x * y * y)
    return y


def _sc_body(ids_hbm, table_hbm, w_hbm, b_hbm, out_hbm,
             idx0, idx1, rows0, rows1, w_v, b_v,
             sem_g0, sem_g1, sem_o0, sem_o1):
    wid = lax.axis_index("s") * NC + lax.axis_index("c")
    base = wid * TPW
    ids_row0 = wid * (TPW // IDS_MINOR)

    pltpu.sync_copy(w_hbm, w_v)
    pltpu.sync_copy(b_hbm, b_v)

    inv_dim = jnp.float32(1.0 / DIM)

    def stage_idx(i, idx_v):
        pltpu.sync_copy(ids_hbm.at[pl.ds(ids_row0 + i * KG, KG)], idx_v)
        for j in range(KG):
            for g in range(GATHER // LANES):
                v = idx_v[j, pl.ds(g * LANES, LANES)]
                idx_v[j, pl.ds(g * LANES, LANES)] = v + v

    def fire_gather(idx_v, rows, sem):
        for j in range(KG):
            pltpu.async_copy(table_hbm.at[idx_v.at[j]],
                             rows.at[pl.ds(j * GATHER, GATHER)], sem)

    def wait_gather(rows, sem):
        pltpu.make_async_copy(table_hbm.at[pl.ds(0, CHUNK)], rows, sem).wait()

    def compute(rows):
        wb = ([w_v[pl.ds(k * LANES, LANES)] for k in range(KV)]
              + [b_v[pl.ds(k * LANES, LANES)] for k in range(KV)])
        UNROLL = 4

        def norm_body(u, wb):
            for tt in range(UNROLL):
                t = u * UNROLL + tt
                vs = [rows[t, pl.ds(k * LANES, LANES)] for k in range(KV)]
                s = (vs[0] + vs[1]) + (vs[2] + vs[3])
                sq = (vs[0] * vs[0] + vs[1] * vs[1]) + (vs[2] * vs[2]
                                                        + vs[3] * vs[3])
                mean = jnp.sum(s) * inv_dim
                msq = jnp.sum(sq) * inv_dim
                var = msq - mean * mean
                rstd = _rsqrt(jnp.maximum(var, 0.0) + jnp.float32(EPS))
                c = -(mean * rstd)
                for k in range(KV):
                    rows[t, pl.ds(k * LANES, LANES)] = (
                        (vs[k] * rstd + c) * wb[k] + wb[KV + k])
            return wb

        lax.fori_loop(0, CHUNK // UNROLL, norm_body, tuple(wb))

    def fire_out(i, rows, sem):
        pltpu.async_copy(rows, out_hbm.at[pl.ds(base + i * CHUNK, CHUNK)],
                         sem)

    def wait_out(rows, sem):
        pltpu.make_async_copy(rows, out_hbm.at[pl.ds(0, CHUNK)], sem).wait()

    stage_idx(0, idx0)
    fire_gather(idx0, rows0, sem_g0)
    stage_idx(1, idx1)
    fire_gather(idx1, rows1, sem_g1)

    def body(h, _):
        i0 = 2 * h
        i1 = 2 * h + 1

        wait_gather(rows0, sem_g0)
        compute(rows0)
        fire_out(i0, rows0, sem_o0)

        @pl.when(h < NCHUNK // 2 - 1)
        def _():
            stage_idx(i0 + 2, idx0)
            wait_out(rows0, sem_o0)
            fire_gather(idx0, rows0, sem_g0)

        wait_gather(rows1, sem_g1)
        compute(rows1)
        fire_out(i1, rows1, sem_o1)

        @pl.when(h < NCHUNK // 2 - 1)
        def _():
            stage_idx(i1 + 2, idx1)
            wait_out(rows1, sem_o1)
            fire_gather(idx1, rows1, sem_g1)
        return 0

    lax.fori_loop(0, NCHUNK // 2, body, 0)
    wait_out(rows0, sem_o0)
    wait_out(rows1, sem_o1)


@jax.jit
def _sc_embed_ln(ids2d, table_view, ln_weight, ln_bias):
    mesh = plsc.VectorSubcoreMesh(
        core_axis_name="c", subcore_axis_name="s",
        num_cores=NC, num_subcores=NS)
    return pl.kernel(
        _sc_body,
        out_type=jax.ShapeDtypeStruct((TOK, DIM), jnp.float32),
        mesh=mesh,
        compiler_params=pltpu.CompilerParams(
            needs_layout_passes=False, use_tc_tiling_on_sc=False),
        scratch_types=[
            pltpu.VMEM((KG, GATHER), jnp.int32),
            pltpu.VMEM((KG, GATHER), jnp.int32),
            pltpu.VMEM((CHUNK, DIM), jnp.float32),
            pltpu.VMEM((CHUNK, DIM), jnp.float32),
            pltpu.VMEM((DIM,), jnp.float32),
            pltpu.VMEM((DIM,), jnp.float32),
            pltpu.SemaphoreType.DMA,
            pltpu.SemaphoreType.DMA,
            pltpu.SemaphoreType.DMA,
            pltpu.SemaphoreType.DMA,
        ],
    )(ids2d, table_view, ln_weight, ln_bias)


def kernel(input_ids, table, ln_weight, ln_bias):
    table_pad = jnp.pad(table, ((0, 0), (0, PADW - DIM)))
    table_view = table_pad.reshape(2 * VOCAB, DIM)
    ids2d = input_ids.astype(jnp.int32).reshape(TOK // IDS_MINOR, IDS_MINOR)
    out = _sc_embed_ln(ids2d, table_view, ln_weight, ln_bias)
    return out.reshape(B, L, DIM)


# speedup vs baseline: 1.5885x; 1.1584x over previous
import functools

import jax
import jax.numpy as jnp
from jax import lax
from jax.experimental import pallas as pl
from jax.experimental.pallas import tpu as pltpu
from jax.experimental.pallas import tpu_sc as plsc

VOCAB = 1000000
DIM = 64
B = 4096
L = 200
EPS = 1e-12

NC = 2
NS = 16
LANES = 16
NW = NC * NS
TOK = B * L
TPW = TOK // NW
CHUNK = 512
GATHER = 128
KG = CHUNK // GATHER
NCHUNK = TPW // CHUNK
IDS_MINOR = 128
PADW = 128
KV = DIM // LANES


def _rsqrt(x):
    i = lax.bitcast_convert_type(x, jnp.int32)
    y = lax.bitcast_convert_type(
        jnp.int32(0x5F3759DF) - lax.shift_right_logical(i, 1), jnp.float32)
    for _ in range(3):
        y = y * (1.5 - 0.5 * x * y * y)
    return y


def _sc_body(ids_hbm, table_hbm, w_hbm, b_hbm, out_hbm,
             idx0, idx1, rows0, rows1, obuf, w_v, b_v,
             sem_g0, sem_g1, sem_o0):
    wid = lax.axis_index("s") * NC + lax.axis_index("c")
    base = wid * TPW
    ids_row0 = wid * (TPW // IDS_MINOR)

    pltpu.sync_copy(w_hbm, w_v)
    pltpu.sync_copy(b_hbm, b_v)

    inv_dim = jnp.float32(1.0 / DIM)

    def stage_idx(i, idx_v):
        pltpu.sync_copy(ids_hbm.at[pl.ds(ids_row0 + i * KG, KG)], idx_v)
        for j in range(KG):
            for g in range(GATHER // LANES):
                v = idx_v[j, pl.ds(g * LANES, LANES)]
                idx_v[j, pl.ds(g * LANES, LANES)] = v + v

    def fire_gather(idx_v, rows, sem):
        for j in range(KG):
            pltpu.async_copy(table_hbm.at[idx_v.at[j]],
                             rows.at[pl.ds(j * GATHER, GATHER)], sem)

    def wait_gather(rows, sem):
        pltpu.make_async_copy(table_hbm.at[pl.ds(0, CHUNK)], rows, sem).wait()

    def compute(rows):
        wb = ([w_v[pl.ds(k * LANES, LANES)] for k in range(KV)]
              + [b_v[pl.ds(k * LANES, LANES)] for k in range(KV)])

        def one(t, off, orow, wb):
            vs = [rows[t, pl.ds(k * LANES, LANES)] for k in range(KV)]
            s = (vs[0] + vs[1]) + (vs[2] + vs[3])
            sq = (vs[0] * vs[0] + vs[1] * vs[1]) + (vs[2] * vs[2]
                                                    + vs[3] * vs[3])
            mean = jnp.sum(s) * inv_dim
            msq = jnp.sum(sq) * inv_dim
            var = msq - mean * mean
            rstd = _rsqrt(jnp.maximum(var, 0.0) + jnp.float32(EPS))
            c = -(mean * rstd)
            for k in range(KV):
                obuf[orow, pl.ds(off + k * LANES, LANES)] = (
                    (vs[k] * rstd + c) * wb[k] + wb[KV + k])

        def norm_body(u, wb):
            for pp in range(2):
                orow = u * 2 + pp
                one(orow * 2, 0, orow, wb)
                one(orow * 2 + 1, DIM, orow, wb)
            return wb

        lax.fori_loop(0, CHUNK // 4, norm_body, tuple(wb))

    OROWS = CHUNK // 2
    obase = base // 2

    def fire_out(i, sem):
        pltpu.async_copy(obuf, out_hbm.at[pl.ds(obase + i * OROWS, OROWS)],
                         sem)

    def wait_out(sem):
        pltpu.make_async_copy(obuf, out_hbm.at[pl.ds(0, OROWS)], sem).wait()

    stage_idx(0, idx0)
    fire_gather(idx0, rows0, sem_g0)
    stage_idx(1, idx1)
    fire_gather(idx1, rows1, sem_g1)

    def body(h, _):
        i0 = 2 * h
        i1 = 2 * h + 1

        wait_gather(rows0, sem_g0)

        @pl.when(h > 0)
        def _():
            wait_out(sem_o0)
        compute(rows0)
        fire_out(i0, sem_o0)

        @pl.when(h < NCHUNK // 2 - 1)
        def _():
            stage_idx(i0 + 2, idx0)
            fire_gather(idx0, rows0, sem_g0)

        wait_gather(rows1, sem_g1)
        wait_out(sem_o0)
        compute(rows1)
        fire_out(i1, sem_o0)

        @pl.when(h < NCHUNK // 2 - 1)
        def _():
            stage_idx(i1 + 2, idx1)
            fire_gather(idx1, rows1, sem_g1)
        return 0

    lax.fori_loop(0, NCHUNK // 2, body, 0)
    wait_out(sem_o0)


@jax.jit
def _sc_embed_ln(ids2d, table_view, ln_weight, ln_bias):
    mesh = plsc.VectorSubcoreMesh(
        core_axis_name="c", subcore_axis_name="s",
        num_cores=NC, num_subcores=NS)
    return pl.kernel(
        _sc_body,
        out_type=jax.ShapeDtypeStruct((TOK // 2, 2 * DIM), jnp.float32),
        mesh=mesh,
        compiler_params=pltpu.CompilerParams(
            needs_layout_passes=False, use_tc_tiling_on_sc=False),
        scratch_types=[
            pltpu.VMEM((KG, GATHER), jnp.int32),
            pltpu.VMEM((KG, GATHER), jnp.int32),
            pltpu.VMEM((CHUNK, DIM), jnp.float32),
            pltpu.VMEM((CHUNK, DIM), jnp.float32),
            pltpu.VMEM((CHUNK // 2, 2 * DIM), jnp.float32),
            pltpu.VMEM((DIM,), jnp.float32),
            pltpu.VMEM((DIM,), jnp.float32),
            pltpu.SemaphoreType.DMA,
            pltpu.SemaphoreType.DMA,
            pltpu.SemaphoreType.DMA,
        ],
    )(ids2d, table_view, ln_weight, ln_bias)


def kernel(input_ids, table, ln_weight, ln_bias):
    table_pad = jnp.pad(table, ((0, 0), (0, PADW - DIM)))
    table_view = table_pad.reshape(2 * VOCAB, DIM)
    ids2d = input_ids.astype(jnp.int32).reshape(TOK // IDS_MINOR, IDS_MINOR)
    out = _sc_embed_ln(ids2d, table_view, ln_weight, ln_bias)
    return out.reshape(B, L, DIM)
